# baseline (speedup 1.0000x reference)
import functools

import jax
import jax.numpy as jnp
from jax import lax
from jax.experimental import pallas as pl
from jax.experimental.pallas import tpu as pltpu
from jax.experimental.pallas import tpu_sc as plsc

N = 10000
E = 320000
NP = 10240
EP = 331776
NBATCH = 81
NC = 2
NS = 16
LANES = 16

_MESH = plsc.VectorSubcoreMesh(
    core_axis_name="c", subcore_axis_name="s", num_cores=NC, num_subcores=NS)
_SC_PARAMS = pltpu.CompilerParams(
    use_tc_tiling_on_sc=False, needs_layout_passes=False)


def _leaky_exp(u, v):
    e = u + v
    return jnp.exp(jnp.maximum(e, 0.2 * e))


def _make_pass1():
    CB = 3
    NCH = NBATCH // CB

    @functools.partial(
        pl.kernel,
        mesh=_MESH,
        compiler_params=_SC_PARAMS,
        out_type=[
            jax.ShapeDtypeStruct((NC * NS, NP * 8), jnp.float32),
            jax.ShapeDtypeStruct((EP // 128, 128, 16), jnp.float32),
        ],
        scratch_types=[
            pltpu.VMEM((NBATCH, 128), jnp.int32),
            pltpu.VMEM((NBATCH, 128), jnp.int32),
            pltpu.VMEM((CB, 128, 16), jnp.float32),
            pltpu.VMEM((CB, 128, 16), jnp.float32),
            pltpu.VMEM((NP * 8,), jnp.float32),
            pltpu.SemaphoreType.DMA,
        ],
    )
    def pass1(srcb, dstb, ast, adt, s_out, w_out, srcv, dstv, gs, gd, s_loc,
              sem):
        cid = lax.axis_index("c")
        sid = lax.axis_index("s")
        wid = cid * NS + sid

        pltpu.sync_copy(srcb.at[wid], srcv)
        pltpu.sync_copy(dstb.at[wid], dstv)

        z16 = jnp.zeros((LANES,), jnp.float32)

        def zrow(i, _):
            s_loc[pl.ds(i * LANES, LANES)] = z16
            return 0
        lax.fori_loop(0, NP * 8 // LANES, zrow, 0)

        lanes = lax.iota(jnp.int32, LANES)
        msk = lanes < 8

        def chunk(ch, _):
            b0 = ch * CB
            descs = []
            for j in range(CB):
                descs.append(pltpu.async_copy(
                    ast.at[srcv.at[b0 + j]], gs.at[j], sem))
                descs.append(pltpu.async_copy(
                    adt.at[dstv.at[b0 + j]], gd.at[j], sem))
            for d in descs:
                d.wait()
            for j in range(CB):
                jv = jnp.full((LANES,), b0 + j, jnp.int32)

                def edge(ii):
                    w = _leaky_exp(gs[j, ii, :], gd[j, ii, :])
                    gs[j, ii, :] = w
                    db = plsc.load_gather(
                        dstv, [jv, jnp.full((LANES,), ii, jnp.int32)])
                    plsc.addupdate_scatter(
                        s_loc, [db * 8 + lanes], w, mask=msk)
                plsc.parallel_loop(0, 128, unroll=8)(edge)
            for j in range(CB):
                pltpu.sync_copy(gs.at[j], w_out.at[wid * NBATCH + b0 + j])
            return 0

        lax.fori_loop(0, NCH, chunk, 0)
        pltpu.sync_copy(s_loc, s_out.at[wid])

    return pass1


def _make_pass2_half(hsel):
    CB = 3
    NCH = NBATCH // CB
    HW = 64
    NV = 4
    ZCH = 2

    @functools.partial(
        pl.kernel,
        mesh=_MESH,
        compiler_params=_SC_PARAMS,
        out_type=jax.ShapeDtypeStruct((NC, NP, HW), jnp.float32),
        scratch_types=[
            pltpu.VMEM((NBATCH, 128), jnp.int32),
            pltpu.VMEM((NBATCH, 128), jnp.int32),
            pltpu.VMEM((CB, 128, 16), jnp.float32),
            pltpu.VMEM((CB, 128, HW), jnp.float32),
            pltpu.VMEM((NP // (NS * ZCH), HW), jnp.float32),
            pltpu.VMEM_SHARED((NP, HW), jnp.float32),
            pltpu.SemaphoreType.DMA,
        ],
    )
    def pass2(srcb, dstb, wt, ht, o_out, srcv, dstv, gw, gh, stage,
              o_acc, sem):
        cid = lax.axis_index("c")
        sid = lax.axis_index("s")
        wid = cid * NS + sid
        rows = NP // (NS * ZCH)

        pltpu.sync_copy(srcb.at[wid], srcv)
        pltpu.sync_copy(dstb.at[wid], dstv)

        def zrow(i, _):
            for h in range(NV):
                stage[i, pl.ds(h * LANES, LANES)] = jnp.zeros(
                    (LANES,), jnp.float32)
            return 0
        lax.fori_loop(0, rows, zrow, 0)
        for z in range(ZCH):
            pltpu.sync_copy(
                stage, o_acc.at[pl.ds((sid * ZCH + z) * rows, rows)])
        plsc.subcore_barrier()

        def chunk(ch, _):
            b0 = ch * CB
            descs = []
            for j in range(CB):
                descs.append(pltpu.async_copy(
                    wt.at[wid * NBATCH + b0 + j], gw.at[j], sem))
                descs.append(pltpu.async_copy(
                    ht.at[srcv.at[b0 + j]], gh.at[j], sem))
            for d in descs:
                d.wait()
            for j in range(CB):
                jv = jnp.full((LANES,), j, jnp.int32)

                def edge(ii):
                    iv = jnp.full((LANES,), ii, jnp.int32)
                    for h in range(NV):
                        ab = plsc.load_gather(
                            gw, [jv, iv,
                                 jnp.full((LANES,), hsel * NV + h,
                                          jnp.int32)])
                        gh[j, ii, pl.ds(h * LANES, LANES)] = (
                            gh[j, ii, pl.ds(h * LANES, LANES)] * ab)
                plsc.parallel_loop(0, 128, unroll=8)(edge)
            for j in range(CB):
                pltpu.sync_copy(gh.at[j], o_acc.at[dstv.at[b0 + j]],
                                add=True)
            return 0

        lax.fori_loop(0, NCH, chunk, 0)
        plsc.subcore_barrier()
        for z in range(ZCH):
            r0 = (sid * ZCH + z) * rows
            pltpu.sync_copy(o_acc.at[pl.ds(r0, rows)], stage)
            pltpu.sync_copy(stage, o_out.at[cid, pl.ds(r0, rows)])

    return pass2


def _make_pass2_l2():
    CB = 9
    NCH = NBATCH // CB
    RW = 16

    @functools.partial(
        pl.kernel,
        mesh=_MESH,
        compiler_params=_SC_PARAMS,
        out_type=jax.ShapeDtypeStruct((NC, NP, RW), jnp.float32),
        scratch_types=[
            pltpu.VMEM((NBATCH, 128), jnp.int32),
            pltpu.VMEM((NBATCH, 128), jnp.int32),
            pltpu.VMEM((CB, 128, 16), jnp.float32),
            pltpu.VMEM((CB, 128, RW), jnp.float32),
            pltpu.VMEM((NP // NS, RW), jnp.float32),
            pltpu.VMEM_SHARED((NP, RW), jnp.float32),
            pltpu.SemaphoreType.DMA,
        ],
    )
    def pass2(srcb, dstb, wt, ht, o_out, srcv, dstv, gw, gh, stage, o_acc,
              sem):
        cid = lax.axis_index("c")
        sid = lax.axis_index("s")
        wid = cid * NS + sid
        rows = NP // NS

        pltpu.sync_copy(srcb.at[wid], srcv)
        pltpu.sync_copy(dstb.at[wid], dstv)

        def zrow(i, _):
            stage[i, :] = jnp.zeros((LANES,), jnp.float32)
            return 0
        lax.fori_loop(0, rows, zrow, 0)
        pltpu.sync_copy(stage, o_acc.at[pl.ds(sid * rows, rows)])
        plsc.subcore_barrier()

        def chunk(ch, _):
            b0 = ch * CB
            descs = []
            for j in range(CB):
                descs.append(pltpu.async_copy(
                    wt.at[wid * NBATCH + b0 + j], gw.at[j], sem))
                descs.append(pltpu.async_copy(
                    ht.at[srcv.at[b0 + j]], gh.at[j], sem))
            for d in descs:
                d.wait()
            for j in range(CB):
                jv = jnp.full((LANES,), j, jnp.int32)
                zv = jnp.zeros((LANES,), jnp.int32)

                def edge(ii):
                    ab = plsc.load_gather(
                        gw, [jv, jnp.full((LANES,), ii, jnp.int32), zv])
                    gh[j, ii, :] = gh[j, ii, :] * ab
                plsc.parallel_loop(0, 128, unroll=8)(edge)
            for j in range(CB):
                pltpu.sync_copy(gh.at[j], o_acc.at[dstv.at[b0 + j]],
                                add=True)
            return 0

        lax.fori_loop(0, NCH, chunk, 0)
        plsc.subcore_barrier()
        pltpu.sync_copy(o_acc.at[pl.ds(sid * rows, rows)], stage)
        pltpu.sync_copy(stage, o_out.at[cid, pl.ds(sid * rows, rows)])

    return pass2


_sc_pass1 = _make_pass1()
_sc_pass2_l1a = _make_pass2_half(0)
_sc_pass2_l1b = _make_pass2_half(1)
_sc_pass2_l2 = _make_pass2_l2()


def _tc1_body(x_ref, w1_ref, as_ref, ad_ref, hlo_ref, hhi_ref, ast_ref,
              adt_ref):
    h = jnp.dot(x_ref[...], w1_ref[...], preferred_element_type=jnp.float32)
    hlo_ref[...] = h[:, :64]
    hhi_ref[...] = h[:, 64:]
    ast_ref[...] = jnp.dot(h, as_ref[...], preferred_element_type=jnp.float32)
    adt_ref[...] = jnp.dot(h, ad_ref[...], preferred_element_type=jnp.float32)


def _tc1(x_pad, W1, As16, Ad16):
    blk = 1024
    return pl.pallas_call(
        _tc1_body,
        grid=(NP // blk,),
        in_specs=[
            pl.BlockSpec((blk, 128), lambda i: (i, 0)),
            pl.BlockSpec((128, 128), lambda i: (0, 0)),
            pl.BlockSpec((128, 16), lambda i: (0, 0)),
            pl.BlockSpec((128, 16), lambda i: (0, 0)),
        ],
        out_specs=[
            pl.BlockSpec((blk, 64), lambda i: (i, 0)),
            pl.BlockSpec((blk, 64), lambda i: (i, 0)),
            pl.BlockSpec((blk, 16), lambda i: (i, 0)),
            pl.BlockSpec((blk, 16), lambda i: (i, 0)),
        ],
        out_shape=[
            jax.ShapeDtypeStruct((NP, 64), jnp.float32),
            jax.ShapeDtypeStruct((NP, 64), jnp.float32),
            jax.ShapeDtypeStruct((NP, 16), jnp.float32),
            jax.ShapeDtypeStruct((NP, 16), jnp.float32),
        ],
    )(x_pad, W1, As16, Ad16)


def _elu(x):
    return jnp.where(x > 0, x, jnp.exp(x) - 1.0)


def _tc2_body(oa_ref, ob_ref, s_ref, b1_ref, w2_ref, as_ref, ad_ref, h2_ref,
              ast_ref, adt_ref):
    o = jnp.concatenate([oa_ref[0] + oa_ref[1], ob_ref[0] + ob_ref[1]],
                        axis=-1)
    blk = o.shape[0]
    r = 1.0 / (jnp.sum(s_ref[...], axis=0) + 1e-16)
    rexp = jnp.broadcast_to(r[:, :, None], (blk, 8, 16)).reshape(blk, 128)
    hb = _elu(o * rexp + b1_ref[...])
    h2 = jnp.dot(hb, w2_ref[...], preferred_element_type=jnp.float32)
    h2_ref[...] = h2
    ast_ref[...] = jnp.dot(h2, as_ref[...], preferred_element_type=jnp.float32)
    adt_ref[...] = jnp.dot(h2, ad_ref[...], preferred_element_type=jnp.float32)


def _tc2(o_a, o_b, s1p, b1r, W2, As2, Ad2):
    blk = 1024
    return pl.pallas_call(
        _tc2_body,
        grid=(NP // blk,),
        in_specs=[
            pl.BlockSpec((2, blk, 64), lambda i: (0, i, 0)),
            pl.BlockSpec((2, blk, 64), lambda i: (0, i, 0)),
            pl.BlockSpec((NC * NS, blk, 8), lambda i: (0, i, 0)),
            pl.BlockSpec((1, 128), lambda i: (0, 0)),
            pl.BlockSpec((128, 16), lambda i: (0, 0)),
            pl.BlockSpec((16, 16), lambda i: (0, 0)),
            pl.BlockSpec((16, 16), lambda i: (0, 0)),
        ],
        out_specs=[
            pl.BlockSpec((blk, 16), lambda i: (i, 0)),
            pl.BlockSpec((blk, 16), lambda i: (i, 0)),
            pl.BlockSpec((blk, 16), lambda i: (i, 0)),
        ],
        out_shape=[
            jax.ShapeDtypeStruct((NP, 16), jnp.float32),
            jax.ShapeDtypeStruct((NP, 16), jnp.float32),
            jax.ShapeDtypeStruct((NP, 16), jnp.float32),
        ],
    )(o_a, o_b, s1p, b1r, W2, As2, Ad2)


def _tc3_body(o_ref, s_ref, b2_ref, wo_ref, bo_ref, out_ref):
    o = o_ref[0] + o_ref[1]
    r = 1.0 / (jnp.sum(s_ref[...], axis=0)[:, 0:1] + 1e-16)
    hb = _elu(o * r + b2_ref[...])
    out_ref[...] = jnp.dot(
        hb, wo_ref[...], preferred_element_type=jnp.float32) + bo_ref[...]


def _tc3(o_part, s2p, b2r, Wo, bor):
    blk = 1024
    return pl.pallas_call(
        _tc3_body,
        grid=(NP // blk,),
        in_specs=[
            pl.BlockSpec((2, blk, 16), lambda i: (0, i, 0)),
            pl.BlockSpec((NC * NS, blk, 8), lambda i: (0, i, 0)),
            pl.BlockSpec((1, 16), lambda i: (0, 0)),
            pl.BlockSpec((16, 16), lambda i: (0, 0)),
            pl.BlockSpec((1, 16), lambda i: (0, 0)),
        ],
        out_specs=pl.BlockSpec((blk, 16), lambda i: (i, 0)),
        out_shape=jax.ShapeDtypeStruct((NP, 16), jnp.float32),
    )(o_part, s2p, b2r, Wo, bor)


def kernel(x, edge_index, W1, a_src1, a_dst1, b1, W2, a_src2, a_dst2, b2,
           Wo, bo):
    f32 = jnp.float32
    x_pad = jnp.pad(x, ((0, NP - N), (0, 0)))
    loops = jnp.arange(N, dtype=jnp.int32)
    fill = jnp.full((EP - E - N,), N, jnp.int32)
    srcb = jnp.concatenate([edge_index[0].astype(jnp.int32), loops, fill]
                           ).reshape(NC * NS, NBATCH, 128)
    dstb = jnp.concatenate([edge_index[1].astype(jnp.int32), loops, fill]
                           ).reshape(NC * NS, NBATCH, 128)

    rows = jnp.arange(128)
    As16 = jnp.zeros((128, 16), f32).at[rows, rows // 16].set(
        a_src1.reshape(128))
    Ad16 = jnp.zeros((128, 16), f32).at[rows, rows // 16].set(
        a_dst1.reshape(128))
    As2 = jnp.zeros((16, 16), f32).at[:, 0].set(a_src2[0])
    Ad2 = jnp.zeros((16, 16), f32).at[:, 0].set(a_dst2[0])

    hlo, hhi, ast1, adt1 = _tc1(x_pad, W1, As16, Ad16)
    s1p, w1t = _sc_pass1(srcb, dstb, ast1, adt1)
    o1a = _sc_pass2_l1a(srcb, dstb, w1t, hlo)
    o1b = _sc_pass2_l1b(srcb, dstb, w1t, hhi)
    h2, ast2, adt2 = _tc2(o1a, o1b, s1p.reshape(NC * NS, NP, 8),
                          b1.reshape(1, 128), W2, As2, Ad2)
    s2p, w2t = _sc_pass1(srcb, dstb, ast2, adt2)
    o2p = _sc_pass2_l2(srcb, dstb, w2t, h2)
    out_pad = _tc3(o2p, s2p.reshape(NC * NS, NP, 8),
                   b2.reshape(1, 16), Wo, bo.reshape(1, 16))
    return out_pad[:N]


# speedup vs baseline: 1.0388x; 1.0388x over previous
import functools

import jax
import jax.numpy as jnp
from jax import lax
from jax._src.lax import slicing as _lsl
from jax.experimental import pallas as pl
from jax.experimental.pallas import tpu as pltpu
from jax.experimental.pallas import tpu_sc as plsc

N = 10000
E = 320000
NP = 10240
EP = 331776
NBATCH = 81
NC = 2
NS = 16
LANES = 16

_MESH = plsc.VectorSubcoreMesh(
    core_axis_name="c", subcore_axis_name="s", num_cores=NC, num_subcores=NS)
_SC_PARAMS = pltpu.CompilerParams(
    use_tc_tiling_on_sc=False, needs_layout_passes=False)


def _leaky_exp(u, v):
    e = u + v
    return jnp.exp(jnp.maximum(e, 0.2 * e))


_GDN = _lsl.GatherDimensionNumbers(
    offset_dims=(), collapsed_slice_dims=(0,), start_index_map=(0,))


def _lane_bcast(vec, lane):
    return _lsl.gather(
        vec, jnp.full((LANES, 1), lane, jnp.int32), _GDN, (1,),
        mode=_lsl.GatherScatterMode.PROMISE_IN_BOUNDS)


def _make_pass1():
    CB = 3
    NCH = NBATCH // CB

    @functools.partial(
        pl.kernel,
        mesh=_MESH,
        compiler_params=_SC_PARAMS,
        out_type=[
            jax.ShapeDtypeStruct((NC * NS, NP * 8), jnp.float32),
            jax.ShapeDtypeStruct((EP // 128, 128, 16), jnp.float32),
        ],
        scratch_types=[
            pltpu.VMEM((NBATCH, 128), jnp.int32),
            pltpu.VMEM((NBATCH, 128), jnp.int32),
            pltpu.VMEM((CB, 128, 16), jnp.float32),
            pltpu.VMEM((CB, 128, 16), jnp.float32),
            pltpu.VMEM((NP * 8,), jnp.float32),
            pltpu.SemaphoreType.DMA,
        ],
    )
    def pass1(srcb, dstb, ast, adt, s_out, w_out, srcv, dstv, gs, gd, s_loc,
              sem):
        cid = lax.axis_index("c")
        sid = lax.axis_index("s")
        wid = cid * NS + sid

        pltpu.sync_copy(srcb.at[wid], srcv)
        pltpu.sync_copy(dstb.at[wid], dstv)

        z16 = jnp.zeros((LANES,), jnp.float32)

        def zrow(i, _):
            s_loc[pl.ds(i * LANES, LANES)] = z16
            return 0
        lax.fori_loop(0, NP * 8 // LANES, zrow, 0)

        lanes = lax.iota(jnp.int32, LANES)
        msk = lanes < 8

        def chunk(ch, _):
            b0 = ch * CB
            descs = []
            for j in range(CB):
                descs.append(pltpu.async_copy(
                    ast.at[srcv.at[b0 + j]], gs.at[j], sem))
                descs.append(pltpu.async_copy(
                    adt.at[dstv.at[b0 + j]], gd.at[j], sem))
            for d in descs:
                d.wait()
            for j in range(CB):
                jv = jnp.full((LANES,), b0 + j, jnp.int32)

                def edge(ii):
                    w = _leaky_exp(gs[j, ii, :], gd[j, ii, :])
                    gs[j, ii, :] = w
                    db = plsc.load_gather(
                        dstv, [jv, jnp.full((LANES,), ii, jnp.int32)])
                    plsc.addupdate_scatter(
                        s_loc, [db * 8 + lanes], w, mask=msk)
                plsc.parallel_loop(0, 128, unroll=4)(edge)
            for j in range(CB):
                pltpu.sync_copy(gs.at[j], w_out.at[wid * NBATCH + b0 + j])
            return 0

        lax.fori_loop(0, NCH, chunk, 0)
        pltpu.sync_copy(s_loc, s_out.at[wid])

    return pass1


def _make_pass2_half(hsel):
    CB = 3
    NCH = NBATCH // CB
    HW = 64
    NV = 4
    ZCH = 2

    @functools.partial(
        pl.kernel,
        mesh=_MESH,
        compiler_params=_SC_PARAMS,
        out_type=jax.ShapeDtypeStruct((NC, NP, HW), jnp.float32),
        scratch_types=[
            pltpu.VMEM((NBATCH, 128), jnp.int32),
            pltpu.VMEM((NBATCH, 128), jnp.int32),
            pltpu.VMEM((CB, 128, 16), jnp.float32),
            pltpu.VMEM((CB, 128, HW), jnp.float32),
            pltpu.VMEM((NP // (NS * ZCH), HW), jnp.float32),
            pltpu.VMEM_SHARED((NP, HW), jnp.float32),
            pltpu.SemaphoreType.DMA,
        ],
    )
    def pass2(srcb, dstb, wt, ht, o_out, srcv, dstv, gw, gh, stage,
              o_acc, sem):
        cid = lax.axis_index("c")
        sid = lax.axis_index("s")
        wid = cid * NS + sid
        rows = NP // (NS * ZCH)

        pltpu.sync_copy(srcb.at[wid], srcv)
        pltpu.sync_copy(dstb.at[wid], dstv)

        def zrow(i, _):
            for h in range(NV):
                stage[i, pl.ds(h * LANES, LANES)] = jnp.zeros(
                    (LANES,), jnp.float32)
            return 0
        lax.fori_loop(0, rows, zrow, 0)
        for z in range(ZCH):
            pltpu.sync_copy(
                stage, o_acc.at[pl.ds((sid * ZCH + z) * rows, rows)])
        plsc.subcore_barrier()

        def chunk(ch, _):
            b0 = ch * CB
            descs = []
            for j in range(CB):
                descs.append(pltpu.async_copy(
                    wt.at[wid * NBATCH + b0 + j], gw.at[j], sem))
                descs.append(pltpu.async_copy(
                    ht.at[srcv.at[b0 + j]], gh.at[j], sem))
            for d in descs:
                d.wait()
            for j in range(CB):
                jv = jnp.full((LANES,), j, jnp.int32)

                def edge(ii):
                    w = gw[j, ii, :]
                    for h in range(NV):
                        ab = _lane_bcast(w, hsel * NV + h)
                        gh[j, ii, pl.ds(h * LANES, LANES)] = (
                            gh[j, ii, pl.ds(h * LANES, LANES)] * ab)
                plsc.parallel_loop(0, 128, unroll=4)(edge)
            for j in range(CB):
                pltpu.sync_copy(gh.at[j], o_acc.at[dstv.at[b0 + j]],
                                add=True)
            return 0

        lax.fori_loop(0, NCH, chunk, 0)
        plsc.subcore_barrier()
        for z in range(ZCH):
            r0 = (sid * ZCH + z) * rows
            pltpu.sync_copy(o_acc.at[pl.ds(r0, rows)], stage)
            pltpu.sync_copy(stage, o_out.at[cid, pl.ds(r0, rows)])

    return pass2


def _make_pass2_l2():
    CB = 9
    NCH = NBATCH // CB
    RW = 16

    @functools.partial(
        pl.kernel,
        mesh=_MESH,
        compiler_params=_SC_PARAMS,
        out_type=jax.ShapeDtypeStruct((NC, NP, RW), jnp.float32),
        scratch_types=[
            pltpu.VMEM((NBATCH, 128), jnp.int32),
            pltpu.VMEM((NBATCH, 128), jnp.int32),
            pltpu.VMEM((CB, 128, 16), jnp.float32),
            pltpu.VMEM((CB, 128, RW), jnp.float32),
            pltpu.VMEM((NP // NS, RW), jnp.float32),
            pltpu.VMEM_SHARED((NP, RW), jnp.float32),
            pltpu.SemaphoreType.DMA,
        ],
    )
    def pass2(srcb, dstb, wt, ht, o_out, srcv, dstv, gw, gh, stage, o_acc,
              sem):
        cid = lax.axis_index("c")
        sid = lax.axis_index("s")
        wid = cid * NS + sid
        rows = NP // NS

        pltpu.sync_copy(srcb.at[wid], srcv)
        pltpu.sync_copy(dstb.at[wid], dstv)

        def zrow(i, _):
            stage[i, :] = jnp.zeros((LANES,), jnp.float32)
            return 0
        lax.fori_loop(0, rows, zrow, 0)
        pltpu.sync_copy(stage, o_acc.at[pl.ds(sid * rows, rows)])
        plsc.subcore_barrier()

        def chunk(ch, _):
            b0 = ch * CB
            descs = []
            for j in range(CB):
                descs.append(pltpu.async_copy(
                    wt.at[wid * NBATCH + b0 + j], gw.at[j], sem))
                descs.append(pltpu.async_copy(
                    ht.at[srcv.at[b0 + j]], gh.at[j], sem))
            for d in descs:
                d.wait()
            for j in range(CB):
                def edge(ii):
                    ab = _lane_bcast(gw[j, ii, :], 0)
                    gh[j, ii, :] = gh[j, ii, :] * ab
                plsc.parallel_loop(0, 128, unroll=4)(edge)
            for j in range(CB):
                pltpu.sync_copy(gh.at[j], o_acc.at[dstv.at[b0 + j]],
                                add=True)
            return 0

        lax.fori_loop(0, NCH, chunk, 0)
        plsc.subcore_barrier()
        pltpu.sync_copy(o_acc.at[pl.ds(sid * rows, rows)], stage)
        pltpu.sync_copy(stage, o_out.at[cid, pl.ds(sid * rows, rows)])

    return pass2


_sc_pass1 = _make_pass1()
_sc_pass2_l1a = _make_pass2_half(0)
_sc_pass2_l1b = _make_pass2_half(1)
_sc_pass2_l2 = _make_pass2_l2()


def _tc1_body(x_ref, w1_ref, as_ref, ad_ref, hlo_ref, hhi_ref, ast_ref,
              adt_ref):
    h = jnp.dot(x_ref[...], w1_ref[...], preferred_element_type=jnp.float32)
    hlo_ref[...] = h[:, :64]
    hhi_ref[...] = h[:, 64:]
    ast_ref[...] = jnp.dot(h, as_ref[...], preferred_element_type=jnp.float32)
    adt_ref[...] = jnp.dot(h, ad_ref[...], preferred_element_type=jnp.float32)


def _tc1(x_pad, W1, As16, Ad16):
    blk = 1024
    return pl.pallas_call(
        _tc1_body,
        grid=(NP // blk,),
        in_specs=[
            pl.BlockSpec((blk, 128), lambda i: (i, 0)),
            pl.BlockSpec((128, 128), lambda i: (0, 0)),
            pl.BlockSpec((128, 16), lambda i: (0, 0)),
            pl.BlockSpec((128, 16), lambda i: (0, 0)),
        ],
        out_specs=[
            pl.BlockSpec((blk, 64), lambda i: (i, 0)),
            pl.BlockSpec((blk, 64), lambda i: (i, 0)),
            pl.BlockSpec((blk, 16), lambda i: (i, 0)),
            pl.BlockSpec((blk, 16), lambda i: (i, 0)),
        ],
        out_shape=[
            jax.ShapeDtypeStruct((NP, 64), jnp.float32),
            jax.ShapeDtypeStruct((NP, 64), jnp.float32),
            jax.ShapeDtypeStruct((NP, 16), jnp.float32),
            jax.ShapeDtypeStruct((NP, 16), jnp.float32),
        ],
    )(x_pad, W1, As16, Ad16)


def _elu(x):
    return jnp.where(x > 0, x, jnp.exp(x) - 1.0)


def _tc2_body(oa_ref, ob_ref, s_ref, b1_ref, w2_ref, as_ref, ad_ref, h2_ref,
              ast_ref, adt_ref):
    o = jnp.concatenate([oa_ref[0] + oa_ref[1], ob_ref[0] + ob_ref[1]],
                        axis=-1)
    blk = o.shape[0]
    r = 1.0 / (jnp.sum(s_ref[...], axis=0) + 1e-16)
    rexp = jnp.broadcast_to(r[:, :, None], (blk, 8, 16)).reshape(blk, 128)
    hb = _elu(o * rexp + b1_ref[...])
    h2 = jnp.dot(hb, w2_ref[...], preferred_element_type=jnp.float32)
    h2_ref[...] = h2
    ast_ref[...] = jnp.dot(h2, as_ref[...], preferred_element_type=jnp.float32)
    adt_ref[...] = jnp.dot(h2, ad_ref[...], preferred_element_type=jnp.float32)


def _tc2(o_a, o_b, s1p, b1r, W2, As2, Ad2):
    blk = 1024
    return pl.pallas_call(
        _tc2_body,
        grid=(NP // blk,),
        in_specs=[
            pl.BlockSpec((2, blk, 64), lambda i: (0, i, 0)),
            pl.BlockSpec((2, blk, 64), lambda i: (0, i, 0)),
            pl.BlockSpec((NC * NS, blk, 8), lambda i: (0, i, 0)),
            pl.BlockSpec((1, 128), lambda i: (0, 0)),
            pl.BlockSpec((128, 16), lambda i: (0, 0)),
            pl.BlockSpec((16, 16), lambda i: (0, 0)),
            pl.BlockSpec((16, 16), lambda i: (0, 0)),
        ],
        out_specs=[
            pl.BlockSpec((blk, 16), lambda i: (i, 0)),
            pl.BlockSpec((blk, 16), lambda i: (i, 0)),
            pl.BlockSpec((blk, 16), lambda i: (i, 0)),
        ],
        out_shape=[
            jax.ShapeDtypeStruct((NP, 16), jnp.float32),
            jax.ShapeDtypeStruct((NP, 16), jnp.float32),
            jax.ShapeDtypeStruct((NP, 16), jnp.float32),
        ],
    )(o_a, o_b, s1p, b1r, W2, As2, Ad2)


def _tc3_body(o_ref, s_ref, b2_ref, wo_ref, bo_ref, out_ref):
    o = o_ref[0] + o_ref[1]
    r = 1.0 / (jnp.sum(s_ref[...], axis=0)[:, 0:1] + 1e-16)
    hb = _elu(o * r + b2_ref[...])
    out_ref[...] = jnp.dot(
        hb, wo_ref[...], preferred_element_type=jnp.float32) + bo_ref[...]


def _tc3(o_part, s2p, b2r, Wo, bor):
    blk = 1024
    return pl.pallas_call(
        _tc3_body,
        grid=(NP // blk,),
        in_specs=[
            pl.BlockSpec((2, blk, 16), lambda i: (0, i, 0)),
            pl.BlockSpec((NC * NS, blk, 8), lambda i: (0, i, 0)),
            pl.BlockSpec((1, 16), lambda i: (0, 0)),
            pl.BlockSpec((16, 16), lambda i: (0, 0)),
            pl.BlockSpec((1, 16), lambda i: (0, 0)),
        ],
        out_specs=pl.BlockSpec((blk, 16), lambda i: (i, 0)),
        out_shape=jax.ShapeDtypeStruct((NP, 16), jnp.float32),
    )(o_part, s2p, b2r, Wo, bor)


def kernel(x, edge_index, W1, a_src1, a_dst1, b1, W2, a_src2, a_dst2, b2,
           Wo, bo):
    f32 = jnp.float32
    x_pad = jnp.pad(x, ((0, NP - N), (0, 0)))
    loops = jnp.arange(N, dtype=jnp.int32)
    fill = jnp.full((EP - E - N,), N, jnp.int32)
    srcb = jnp.concatenate([edge_index[0].astype(jnp.int32), loops, fill]
                           ).reshape(NC * NS, NBATCH, 128)
    dstb = jnp.concatenate([edge_index[1].astype(jnp.int32), loops, fill]
                           ).reshape(NC * NS, NBATCH, 128)

    rows = jnp.arange(128)
    As16 = jnp.zeros((128, 16), f32).at[rows, rows // 16].set(
        a_src1.reshape(128))
    Ad16 = jnp.zeros((128, 16), f32).at[rows, rows // 16].set(
        a_dst1.reshape(128))
    As2 = jnp.zeros((16, 16), f32).at[:, 0].set(a_src2[0])
    Ad2 = jnp.zeros((16, 16), f32).at[:, 0].set(a_dst2[0])

    hlo, hhi, ast1, adt1 = _tc1(x_pad, W1, As16, Ad16)
    s1p, w1t = _sc_pass1(srcb, dstb, ast1, adt1)
    o1a = _sc_pass2_l1a(srcb, dstb, w1t, hlo)
    o1b = _sc_pass2_l1b(srcb, dstb, w1t, hhi)
    h2, ast2, adt2 = _tc2(o1a, o1b, s1p.reshape(NC * NS, NP, 8),
                          b1.reshape(1, 128), W2, As2, Ad2)
    s2p, w2t = _sc_pass1(srcb, dstb, ast2, adt2)
    o2p = _sc_pass2_l2(srcb, dstb, w2t, h2)
    out_pad = _tc3(o2p, s2p.reshape(NC * NS, NP, 8),
                   b2.reshape(1, 16), Wo, bo.reshape(1, 16))
    return out_pad[:N]


# speedup vs baseline: 1.1024x; 1.0613x over previous
import functools

import jax
import jax.numpy as jnp
from jax import lax
from jax._src.lax import slicing as _lsl
from jax.experimental import pallas as pl
from jax.experimental.pallas import tpu as pltpu
from jax.experimental.pallas import tpu_sc as plsc

N = 10000
E = 320000
NP = 10240
EP = 331776
NBATCH = 81
NC = 2
NS = 16
LANES = 16

_MESH = plsc.VectorSubcoreMesh(
    core_axis_name="c", subcore_axis_name="s", num_cores=NC, num_subcores=NS)
_SC_PARAMS = pltpu.CompilerParams(
    use_tc_tiling_on_sc=False, needs_layout_passes=False)


def _leaky_exp(u, v):
    e = u + v
    return jnp.exp(jnp.maximum(e, 0.2 * e))


_GDN = _lsl.GatherDimensionNumbers(
    offset_dims=(), collapsed_slice_dims=(0,), start_index_map=(0,))


def _lane_bcast(vec, lane):
    return _lsl.gather(
        vec, jnp.full((LANES, 1), lane, jnp.int32), _GDN, (1,),
        mode=_lsl.GatherScatterMode.PROMISE_IN_BOUNDS)


def _make_pass1():
    CB = 3
    NCH = NBATCH // CB

    @functools.partial(
        pl.kernel,
        mesh=_MESH,
        compiler_params=_SC_PARAMS,
        out_type=[
            jax.ShapeDtypeStruct((NC * NS, NP * 8), jnp.float32),
            jax.ShapeDtypeStruct((EP // 128, 128, 16), jnp.float32),
        ],
        scratch_types=[
            pltpu.VMEM((NBATCH, 128), jnp.int32),
            pltpu.VMEM((NBATCH, 128), jnp.int32),
            pltpu.VMEM((CB, 128, 16), jnp.float32),
            pltpu.VMEM((CB, 128, 16), jnp.float32),
            pltpu.VMEM((NP * 8,), jnp.float32),
            pltpu.SemaphoreType.DMA,
        ],
    )
    def pass1(srcb, dstb, ast, adt, s_out, w_out, srcv, dstv, gs, gd, s_loc,
              sem):
        cid = lax.axis_index("c")
        sid = lax.axis_index("s")
        wid = cid * NS + sid

        pltpu.sync_copy(srcb.at[wid], srcv)
        pltpu.sync_copy(dstb.at[wid], dstv)

        z16 = jnp.zeros((LANES,), jnp.float32)

        def zrow(i, _):
            s_loc[pl.ds(i * LANES, LANES)] = z16
            return 0
        lax.fori_loop(0, NP * 8 // LANES, zrow, 0)

        lanes = lax.iota(jnp.int32, LANES)
        msk = lanes < 8

        def chunk(ch, _):
            b0 = ch * CB
            descs = []
            for j in range(CB):
                descs.append(pltpu.async_copy(
                    ast.at[srcv.at[b0 + j]], gs.at[j], sem))
                descs.append(pltpu.async_copy(
                    adt.at[dstv.at[b0 + j]], gd.at[j], sem))
            for d in descs:
                d.wait()
            for j in range(CB):
                jv = jnp.full((LANES,), b0 + j, jnp.int32)

                def edge(ii):
                    w = _leaky_exp(gs[j, ii, :], gd[j, ii, :])
                    gs[j, ii, :] = w
                    db = plsc.load_gather(
                        dstv, [jv, jnp.full((LANES,), ii, jnp.int32)])
                    plsc.addupdate_scatter(
                        s_loc, [db * 8 + lanes], w, mask=msk)
                plsc.parallel_loop(0, 128, unroll=4)(edge)
            for j in range(CB):
                pltpu.sync_copy(gs.at[j], w_out.at[wid * NBATCH + b0 + j])
            return 0

        lax.fori_loop(0, NCH, chunk, 0)
        pltpu.sync_copy(s_loc, s_out.at[wid])

    return pass1


def _make_pass2_half(hsel):
    HW = 64
    NV = 4
    ZCH = 4

    @functools.partial(
        pl.kernel,
        mesh=_MESH,
        compiler_params=_SC_PARAMS,
        out_type=jax.ShapeDtypeStruct((NC, NP, HW), jnp.float32),
        scratch_types=[
            pltpu.VMEM((NBATCH, 128), jnp.int32),
            pltpu.VMEM((NBATCH, 128), jnp.int32),
            pltpu.VMEM((2, 128, 16), jnp.float32),
            pltpu.VMEM((2, 128, HW), jnp.float32),
            pltpu.VMEM((NP // (NS * ZCH), HW), jnp.float32),
            pltpu.VMEM_SHARED((NP, HW), jnp.float32),
            pltpu.SemaphoreType.DMA,
        ],
    )
    def pass2(srcb, dstb, wt, ht, o_out, srcv, dstv, gw, gh, stage,
              o_acc, sem):
        cid = lax.axis_index("c")
        sid = lax.axis_index("s")
        wid = cid * NS + sid
        rows = NP // (NS * ZCH)

        pltpu.sync_copy(srcb.at[wid], srcv)
        pltpu.sync_copy(dstb.at[wid], dstv)

        def zrow(i, _):
            for h in range(NV):
                stage[i, pl.ds(h * LANES, LANES)] = jnp.zeros(
                    (LANES,), jnp.float32)
            return 0
        lax.fori_loop(0, rows, zrow, 0)
        for z in range(ZCH):
            pltpu.sync_copy(
                stage, o_acc.at[pl.ds((sid * ZCH + z) * rows, rows)])
        plsc.subcore_barrier()

        def fire(b, p):
            pltpu.async_copy(wt.at[wid * NBATCH + b], gw.at[p], sem)
            pltpu.async_copy(ht.at[srcv.at[b]], gh.at[p], sem)

        def drain(p):
            pltpu.make_async_copy(wt.at[0], gw.at[p], sem).wait()
            pltpu.make_async_copy(ht.at[srcv.at[0]], gh.at[p], sem).wait()

        def work(b, p):
            def edge(ii):
                w = gw[p, ii, :]
                for h in range(NV):
                    ab = _lane_bcast(w, hsel * NV + h)
                    gh[p, ii, pl.ds(h * LANES, LANES)] = (
                        gh[p, ii, pl.ds(h * LANES, LANES)] * ab)
            plsc.parallel_loop(0, 128, unroll=4)(edge)
            pltpu.sync_copy(gh.at[p], o_acc.at[dstv.at[b]], add=True)

        fire(0, 0)

        def it(k, _):
            fire(2 * k + 1, 1)
            drain(0)
            work(2 * k, 0)
            fire(2 * k + 2, 0)
            drain(1)
            work(2 * k + 1, 1)
            return 0

        lax.fori_loop(0, (NBATCH - 1) // 2, it, 0)
        drain(0)
        work(NBATCH - 1, 0)

        plsc.subcore_barrier()
        for z in range(ZCH):
            r0 = (sid * ZCH + z) * rows
            pltpu.sync_copy(o_acc.at[pl.ds(r0, rows)], stage)
            pltpu.sync_copy(stage, o_out.at[cid, pl.ds(r0, rows)])

    return pass2


def _make_pass2_l2():
    CB = 9
    NCH = NBATCH // CB
    RW = 16

    @functools.partial(
        pl.kernel,
        mesh=_MESH,
        compiler_params=_SC_PARAMS,
        out_type=jax.ShapeDtypeStruct((NC, NP, RW), jnp.float32),
        scratch_types=[
            pltpu.VMEM((NBATCH, 128), jnp.int32),
            pltpu.VMEM((NBATCH, 128), jnp.int32),
            pltpu.VMEM((CB, 128, 16), jnp.float32),
            pltpu.VMEM((CB, 128, RW), jnp.float32),
            pltpu.VMEM((NP // NS, RW), jnp.float32),
            pltpu.VMEM_SHARED((NP, RW), jnp.float32),
            pltpu.SemaphoreType.DMA,
        ],
    )
    def pass2(srcb, dstb, wt, ht, o_out, srcv, dstv, gw, gh, stage, o_acc,
              sem):
        cid = lax.axis_index("c")
        sid = lax.axis_index("s")
        wid = cid * NS + sid
        rows = NP // NS

        pltpu.sync_copy(srcb.at[wid], srcv)
        pltpu.sync_copy(dstb.at[wid], dstv)

        def zrow(i, _):
            stage[i, :] = jnp.zeros((LANES,), jnp.float32)
            return 0
        lax.fori_loop(0, rows, zrow, 0)
        pltpu.sync_copy(stage, o_acc.at[pl.ds(sid * rows, rows)])
        plsc.subcore_barrier()

        def chunk(ch, _):
            b0 = ch * CB
            descs = []
            for j in range(CB):
                descs.append(pltpu.async_copy(
                    wt.at[wid * NBATCH + b0 + j], gw.at[j], sem))
                descs.append(pltpu.async_copy(
                    ht.at[srcv.at[b0 + j]], gh.at[j], sem))
            for d in descs:
                d.wait()
            for j in range(CB):
                def edge(ii):
                    ab = _lane_bcast(gw[j, ii, :], 0)
                    gh[j, ii, :] = gh[j, ii, :] * ab
                plsc.parallel_loop(0, 128, unroll=4)(edge)
            for j in range(CB):
                pltpu.sync_copy(gh.at[j], o_acc.at[dstv.at[b0 + j]],
                                add=True)
            return 0

        lax.fori_loop(0, NCH, chunk, 0)
        plsc.subcore_barrier()
        pltpu.sync_copy(o_acc.at[pl.ds(sid * rows, rows)], stage)
        pltpu.sync_copy(stage, o_out.at[cid, pl.ds(sid * rows, rows)])

    return pass2


_sc_pass1 = _make_pass1()
_sc_pass2_l1a = _make_pass2_half(0)
_sc_pass2_l1b = _make_pass2_half(1)
_sc_pass2_l2 = _make_pass2_l2()


def _tc1_body(x_ref, w1_ref, as_ref, ad_ref, hlo_ref, hhi_ref, ast_ref,
              adt_ref):
    h = jnp.dot(x_ref[...], w1_ref[...], preferred_element_type=jnp.float32)
    hlo_ref[...] = h[:, :64]
    hhi_ref[...] = h[:, 64:]
    ast_ref[...] = jnp.dot(h, as_ref[...], preferred_element_type=jnp.float32)
    adt_ref[...] = jnp.dot(h, ad_ref[...], preferred_element_type=jnp.float32)


def _tc1(x_pad, W1, As16, Ad16):
    blk = 1024
    return pl.pallas_call(
        _tc1_body,
        grid=(NP // blk,),
        in_specs=[
            pl.BlockSpec((blk, 128), lambda i: (i, 0)),
            pl.BlockSpec((128, 128), lambda i: (0, 0)),
            pl.BlockSpec((128, 16), lambda i: (0, 0)),
            pl.BlockSpec((128, 16), lambda i: (0, 0)),
        ],
        out_specs=[
            pl.BlockSpec((blk, 64), lambda i: (i, 0)),
            pl.BlockSpec((blk, 64), lambda i: (i, 0)),
            pl.BlockSpec((blk, 16), lambda i: (i, 0)),
            pl.BlockSpec((blk, 16), lambda i: (i, 0)),
        ],
        out_shape=[
            jax.ShapeDtypeStruct((NP, 64), jnp.float32),
            jax.ShapeDtypeStruct((NP, 64), jnp.float32),
            jax.ShapeDtypeStruct((NP, 16), jnp.float32),
            jax.ShapeDtypeStruct((NP, 16), jnp.float32),
        ],
    )(x_pad, W1, As16, Ad16)


def _elu(x):
    return jnp.where(x > 0, x, jnp.exp(x) - 1.0)


def _tc2_body(oa_ref, ob_ref, s_ref, b1_ref, w2_ref, as_ref, ad_ref, h2_ref,
              ast_ref, adt_ref):
    o = jnp.concatenate([oa_ref[0] + oa_ref[1], ob_ref[0] + ob_ref[1]],
                        axis=-1)
    blk = o.shape[0]
    r = 1.0 / (jnp.sum(s_ref[...], axis=0) + 1e-16)
    rexp = jnp.broadcast_to(r[:, :, None], (blk, 8, 16)).reshape(blk, 128)
    hb = _elu(o * rexp + b1_ref[...])
    h2 = jnp.dot(hb, w2_ref[...], preferred_element_type=jnp.float32)
    h2_ref[...] = h2
    ast_ref[...] = jnp.dot(h2, as_ref[...], preferred_element_type=jnp.float32)
    adt_ref[...] = jnp.dot(h2, ad_ref[...], preferred_element_type=jnp.float32)


def _tc2(o_a, o_b, s1p, b1r, W2, As2, Ad2):
    blk = 1024
    return pl.pallas_call(
        _tc2_body,
        grid=(NP // blk,),
        in_specs=[
            pl.BlockSpec((2, blk, 64), lambda i: (0, i, 0)),
            pl.BlockSpec((2, blk, 64), lambda i: (0, i, 0)),
            pl.BlockSpec((NC * NS, blk, 8), lambda i: (0, i, 0)),
            pl.BlockSpec((1, 128), lambda i: (0, 0)),
            pl.BlockSpec((128, 16), lambda i: (0, 0)),
            pl.BlockSpec((16, 16), lambda i: (0, 0)),
            pl.BlockSpec((16, 16), lambda i: (0, 0)),
        ],
        out_specs=[
            pl.BlockSpec((blk, 16), lambda i: (i, 0)),
            pl.BlockSpec((blk, 16), lambda i: (i, 0)),
            pl.BlockSpec((blk, 16), lambda i: (i, 0)),
        ],
        out_shape=[
            jax.ShapeDtypeStruct((NP, 16), jnp.float32),
            jax.ShapeDtypeStruct((NP, 16), jnp.float32),
            jax.ShapeDtypeStruct((NP, 16), jnp.float32),
        ],
    )(o_a, o_b, s1p, b1r, W2, As2, Ad2)


def _tc3_body(o_ref, s_ref, b2_ref, wo_ref, bo_ref, out_ref):
    o = o_ref[0] + o_ref[1]
    r = 1.0 / (jnp.sum(s_ref[...], axis=0)[:, 0:1] + 1e-16)
    hb = _elu(o * r + b2_ref[...])
    out_ref[...] = jnp.dot(
        hb, wo_ref[...], preferred_element_type=jnp.float32) + bo_ref[...]


def _tc3(o_part, s2p, b2r, Wo, bor):
    blk = 1024
    return pl.pallas_call(
        _tc3_body,
        grid=(NP // blk,),
        in_specs=[
            pl.BlockSpec((2, blk, 16), lambda i: (0, i, 0)),
            pl.BlockSpec((NC * NS, blk, 8), lambda i: (0, i, 0)),
            pl.BlockSpec((1, 16), lambda i: (0, 0)),
            pl.BlockSpec((16, 16), lambda i: (0, 0)),
            pl.BlockSpec((1, 16), lambda i: (0, 0)),
        ],
        out_specs=pl.BlockSpec((blk, 16), lambda i: (i, 0)),
        out_shape=jax.ShapeDtypeStruct((NP, 16), jnp.float32),
    )(o_part, s2p, b2r, Wo, bor)


def kernel(x, edge_index, W1, a_src1, a_dst1, b1, W2, a_src2, a_dst2, b2,
           Wo, bo):
    f32 = jnp.float32
    x_pad = jnp.pad(x, ((0, NP - N), (0, 0)))
    loops = jnp.arange(N, dtype=jnp.int32)
    fill = jnp.full((EP - E - N,), N, jnp.int32)
    srcb = jnp.concatenate([edge_index[0].astype(jnp.int32), loops, fill]
                           ).reshape(NC * NS, NBATCH, 128)
    dstb = jnp.concatenate([edge_index[1].astype(jnp.int32), loops, fill]
                           ).reshape(NC * NS, NBATCH, 128)

    rows = jnp.arange(128)
    As16 = jnp.zeros((128, 16), f32).at[rows, rows // 16].set(
        a_src1.reshape(128))
    Ad16 = jnp.zeros((128, 16), f32).at[rows, rows // 16].set(
        a_dst1.reshape(128))
    As2 = jnp.zeros((16, 16), f32).at[:, 0].set(a_src2[0])
    Ad2 = jnp.zeros((16, 16), f32).at[:, 0].set(a_dst2[0])

    hlo, hhi, ast1, adt1 = _tc1(x_pad, W1, As16, Ad16)
    s1p, w1t = _sc_pass1(srcb, dstb, ast1, adt1)
    o1a = _sc_pass2_l1a(srcb, dstb, w1t, hlo)
    o1b = _sc_pass2_l1b(srcb, dstb, w1t, hhi)
    h2, ast2, adt2 = _tc2(o1a, o1b, s1p.reshape(NC * NS, NP, 8),
                          b1.reshape(1, 128), W2, As2, Ad2)
    s2p, w2t = _sc_pass1(srcb, dstb, ast2, adt2)
    o2p = _sc_pass2_l2(srcb, dstb, w2t, h2)
    out_pad = _tc3(o2p, s2p.reshape(NC * NS, NP, 8),
                   b2.reshape(1, 16), Wo, bo.reshape(1, 16))
    return out_pad[:N]


# speedup vs baseline: 1.1239x; 1.0195x over previous
import functools

import jax
import jax.numpy as jnp
from jax import lax
from jax._src.lax import slicing as _lsl
from jax.experimental import pallas as pl
from jax.experimental.pallas import tpu as pltpu
from jax.experimental.pallas import tpu_sc as plsc

N = 10000
E = 320000
NP = 10240
EP = 331776
NBATCH = 81
NC = 2
NS = 16
LANES = 16

_MESH = plsc.VectorSubcoreMesh(
    core_axis_name="c", subcore_axis_name="s", num_cores=NC, num_subcores=NS)
_SC_PARAMS = pltpu.CompilerParams(
    use_tc_tiling_on_sc=False, needs_layout_passes=False)


def _leaky_exp(u, v):
    e = u + v
    return jnp.exp(jnp.maximum(e, 0.2 * e))


_GDN = _lsl.GatherDimensionNumbers(
    offset_dims=(), collapsed_slice_dims=(0,), start_index_map=(0,))


def _lane_bcast(vec, lane):
    return _lsl.gather(
        vec, jnp.full((LANES, 1), lane, jnp.int32), _GDN, (1,),
        mode=_lsl.GatherScatterMode.PROMISE_IN_BOUNDS)


def _make_pass1():
    @functools.partial(
        pl.kernel,
        mesh=_MESH,
        compiler_params=_SC_PARAMS,
        out_type=[
            jax.ShapeDtypeStruct((NC * NS, NP * 8), jnp.float32),
            jax.ShapeDtypeStruct((EP // 128, 128, 16), jnp.float32),
        ],
        scratch_types=[
            pltpu.VMEM((NBATCH, 128), jnp.int32),
            pltpu.VMEM((NBATCH, 128), jnp.int32),
            pltpu.VMEM((2, 128, 16), jnp.float32),
            pltpu.VMEM((2, 128, 16), jnp.float32),
            pltpu.VMEM((NP * 8,), jnp.float32),
            pltpu.SemaphoreType.DMA,
        ],
    )
    def pass1(srcb, dstb, ast, adt, s_out, w_out, srcv, dstv, gs, gd, s_loc,
              sem):
        cid = lax.axis_index("c")
        sid = lax.axis_index("s")
        wid = cid * NS + sid

        pltpu.sync_copy(srcb.at[wid], srcv)
        pltpu.sync_copy(dstb.at[wid], dstv)

        z16 = jnp.zeros((LANES,), jnp.float32)

        def zrow(i, _):
            s_loc[pl.ds(i * LANES, LANES)] = z16
            return 0
        lax.fori_loop(0, NP * 8 // LANES, zrow, 0)

        lanes = lax.iota(jnp.int32, LANES)
        msk = lanes < 8

        def fire(b, p):
            pltpu.async_copy(ast.at[srcv.at[b]], gs.at[p], sem)
            pltpu.async_copy(adt.at[dstv.at[b]], gd.at[p], sem)

        def drain(p):
            pltpu.make_async_copy(ast.at[srcv.at[0]], gs.at[p], sem).wait()
            pltpu.make_async_copy(adt.at[dstv.at[0]], gd.at[p], sem).wait()

        def work(b, p):
            jv = jnp.full((LANES,), b, jnp.int32)

            def edge(ii):
                w = _leaky_exp(gs[p, ii, :], gd[p, ii, :])
                gs[p, ii, :] = w
                db = plsc.load_gather(
                    dstv, [jv, jnp.full((LANES,), ii, jnp.int32)])
                plsc.addupdate_scatter(
                    s_loc, [db * 8 + lanes], w, mask=msk)
            plsc.parallel_loop(0, 128, unroll=4)(edge)
            pltpu.sync_copy(gs.at[p], w_out.at[wid * NBATCH + b])

        fire(0, 0)

        def it(k, _):
            fire(2 * k + 1, 1)
            drain(0)
            work(2 * k, 0)
            fire(2 * k + 2, 0)
            drain(1)
            work(2 * k + 1, 1)
            return 0

        lax.fori_loop(0, (NBATCH - 1) // 2, it, 0)
        drain(0)
        work(NBATCH - 1, 0)
        pltpu.sync_copy(s_loc, s_out.at[wid])

    return pass1


def _make_pass2_half(hsel):
    HW = 64
    NV = 4
    ZCH = 4

    @functools.partial(
        pl.kernel,
        mesh=_MESH,
        compiler_params=_SC_PARAMS,
        out_type=jax.ShapeDtypeStruct((NC, NP, HW), jnp.float32),
        scratch_types=[
            pltpu.VMEM((NBATCH, 128), jnp.int32),
            pltpu.VMEM((NBATCH, 128), jnp.int32),
            pltpu.VMEM((2, 128, 16), jnp.float32),
            pltpu.VMEM((2, 128, HW), jnp.float32),
            pltpu.VMEM((NP // (NS * ZCH), HW), jnp.float32),
            pltpu.VMEM_SHARED((NP, HW), jnp.float32),
            pltpu.SemaphoreType.DMA,
        ],
    )
    def pass2(srcb, dstb, wt, ht, o_out, srcv, dstv, gw, gh, stage,
              o_acc, sem):
        cid = lax.axis_index("c")
        sid = lax.axis_index("s")
        wid = cid * NS + sid
        rows = NP // (NS * ZCH)

        pltpu.sync_copy(srcb.at[wid], srcv)
        pltpu.sync_copy(dstb.at[wid], dstv)

        def zrow(i, _):
            for h in range(NV):
                stage[i, pl.ds(h * LANES, LANES)] = jnp.zeros(
                    (LANES,), jnp.float32)
            return 0
        lax.fori_loop(0, rows, zrow, 0)
        for z in range(ZCH):
            pltpu.sync_copy(
                stage, o_acc.at[pl.ds((sid * ZCH + z) * rows, rows)])
        plsc.subcore_barrier()

        def fire(b, p):
            pltpu.async_copy(wt.at[wid * NBATCH + b], gw.at[p], sem)
            pltpu.async_copy(ht.at[srcv.at[b]], gh.at[p], sem)

        def drain(p):
            pltpu.make_async_copy(wt.at[0], gw.at[p], sem).wait()
            pltpu.make_async_copy(ht.at[srcv.at[0]], gh.at[p], sem).wait()

        def work(b, p):
            def edge(ii):
                w = gw[p, ii, :]
                for h in range(NV):
                    ab = _lane_bcast(w, hsel * NV + h)
                    gh[p, ii, pl.ds(h * LANES, LANES)] = (
                        gh[p, ii, pl.ds(h * LANES, LANES)] * ab)
            plsc.parallel_loop(0, 128, unroll=4)(edge)
            pltpu.sync_copy(gh.at[p], o_acc.at[dstv.at[b]], add=True)

        fire(0, 0)

        def it(k, _):
            fire(2 * k + 1, 1)
            drain(0)
            work(2 * k, 0)
            fire(2 * k + 2, 0)
            drain(1)
            work(2 * k + 1, 1)
            return 0

        lax.fori_loop(0, (NBATCH - 1) // 2, it, 0)
        drain(0)
        work(NBATCH - 1, 0)

        plsc.subcore_barrier()
        for z in range(ZCH):
            r0 = (sid * ZCH + z) * rows
            pltpu.sync_copy(o_acc.at[pl.ds(r0, rows)], stage)
            pltpu.sync_copy(stage, o_out.at[cid, pl.ds(r0, rows)])

    return pass2


def _make_pass2_l2():
    CB = 9
    NCH = NBATCH // CB
    RW = 16

    @functools.partial(
        pl.kernel,
        mesh=_MESH,
        compiler_params=_SC_PARAMS,
        out_type=jax.ShapeDtypeStruct((NC, NP, RW), jnp.float32),
        scratch_types=[
            pltpu.VMEM((NBATCH, 128), jnp.int32),
            pltpu.VMEM((NBATCH, 128), jnp.int32),
            pltpu.VMEM((CB, 128, 16), jnp.float32),
            pltpu.VMEM((CB, 128, RW), jnp.float32),
            pltpu.VMEM((NP // NS, RW), jnp.float32),
            pltpu.VMEM_SHARED((NP, RW), jnp.float32),
            pltpu.SemaphoreType.DMA,
        ],
    )
    def pass2(srcb, dstb, wt, ht, o_out, srcv, dstv, gw, gh, stage, o_acc,
              sem):
        cid = lax.axis_index("c")
        sid = lax.axis_index("s")
        wid = cid * NS + sid
        rows = NP // NS

        pltpu.sync_copy(srcb.at[wid], srcv)
        pltpu.sync_copy(dstb.at[wid], dstv)

        def zrow(i, _):
            stage[i, :] = jnp.zeros((LANES,), jnp.float32)
            return 0
        lax.fori_loop(0, rows, zrow, 0)
        pltpu.sync_copy(stage, o_acc.at[pl.ds(sid * rows, rows)])
        plsc.subcore_barrier()

        def chunk(ch, _):
            b0 = ch * CB
            descs = []
            for j in range(CB):
                descs.append(pltpu.async_copy(
                    wt.at[wid * NBATCH + b0 + j], gw.at[j], sem))
                descs.append(pltpu.async_copy(
                    ht.at[srcv.at[b0 + j]], gh.at[j], sem))
            for d in descs:
                d.wait()
            for j in range(CB):
                def edge(ii):
                    ab = _lane_bcast(gw[j, ii, :], 0)
                    gh[j, ii, :] = gh[j, ii, :] * ab
                plsc.parallel_loop(0, 128, unroll=4)(edge)
            for j in range(CB):
                pltpu.sync_copy(gh.at[j], o_acc.at[dstv.at[b0 + j]],
                                add=True)
            return 0

        lax.fori_loop(0, NCH, chunk, 0)
        plsc.subcore_barrier()
        pltpu.sync_copy(o_acc.at[pl.ds(sid * rows, rows)], stage)
        pltpu.sync_copy(stage, o_out.at[cid, pl.ds(sid * rows, rows)])

    return pass2


_sc_pass1 = _make_pass1()
_sc_pass2_l1a = _make_pass2_half(0)
_sc_pass2_l1b = _make_pass2_half(1)
_sc_pass2_l2 = _make_pass2_l2()


def _tc1_body(x_ref, w1_ref, as_ref, ad_ref, hlo_ref, hhi_ref, ast_ref,
              adt_ref):
    h = jnp.dot(x_ref[...], w1_ref[...], preferred_element_type=jnp.float32)
    hlo_ref[...] = h[:, :64]
    hhi_ref[...] = h[:, 64:]
    ast_ref[...] = jnp.dot(h, as_ref[...], preferred_element_type=jnp.float32)
    adt_ref[...] = jnp.dot(h, ad_ref[...], preferred_element_type=jnp.float32)


def _tc1(x_pad, W1, As16, Ad16):
    blk = 1024
    return pl.pallas_call(
        _tc1_body,
        grid=(NP // blk,),
        in_specs=[
            pl.BlockSpec((blk, 128), lambda i: (i, 0)),
            pl.BlockSpec((128, 128), lambda i: (0, 0)),
            pl.BlockSpec((128, 16), lambda i: (0, 0)),
            pl.BlockSpec((128, 16), lambda i: (0, 0)),
        ],
        out_specs=[
            pl.BlockSpec((blk, 64), lambda i: (i, 0)),
            pl.BlockSpec((blk, 64), lambda i: (i, 0)),
            pl.BlockSpec((blk, 16), lambda i: (i, 0)),
            pl.BlockSpec((blk, 16), lambda i: (i, 0)),
        ],
        out_shape=[
            jax.ShapeDtypeStruct((NP, 64), jnp.float32),
            jax.ShapeDtypeStruct((NP, 64), jnp.float32),
            jax.ShapeDtypeStruct((NP, 16), jnp.float32),
            jax.ShapeDtypeStruct((NP, 16), jnp.float32),
        ],
    )(x_pad, W1, As16, Ad16)


def _elu(x):
    return jnp.where(x > 0, x, jnp.exp(x) - 1.0)


def _tc2_body(oa_ref, ob_ref, s_ref, b1_ref, w2_ref, as_ref, ad_ref, h2_ref,
              ast_ref, adt_ref):
    o = jnp.concatenate([oa_ref[0] + oa_ref[1], ob_ref[0] + ob_ref[1]],
                        axis=-1)
    blk = o.shape[0]
    r = 1.0 / (jnp.sum(s_ref[...], axis=0) + 1e-16)
    rexp = jnp.broadcast_to(r[:, :, None], (blk, 8, 16)).reshape(blk, 128)
    hb = _elu(o * rexp + b1_ref[...])
    h2 = jnp.dot(hb, w2_ref[...], preferred_element_type=jnp.float32)
    h2_ref[...] = h2
    ast_ref[...] = jnp.dot(h2, as_ref[...], preferred_element_type=jnp.float32)
    adt_ref[...] = jnp.dot(h2, ad_ref[...], preferred_element_type=jnp.float32)


def _tc2(o_a, o_b, s1p, b1r, W2, As2, Ad2):
    blk = 1024
    return pl.pallas_call(
        _tc2_body,
        grid=(NP // blk,),
        in_specs=[
            pl.BlockSpec((2, blk, 64), lambda i: (0, i, 0)),
            pl.BlockSpec((2, blk, 64), lambda i: (0, i, 0)),
            pl.BlockSpec((NC * NS, blk, 8), lambda i: (0, i, 0)),
            pl.BlockSpec((1, 128), lambda i: (0, 0)),
            pl.BlockSpec((128, 16), lambda i: (0, 0)),
            pl.BlockSpec((16, 16), lambda i: (0, 0)),
            pl.BlockSpec((16, 16), lambda i: (0, 0)),
        ],
        out_specs=[
            pl.BlockSpec((blk, 16), lambda i: (i, 0)),
            pl.BlockSpec((blk, 16), lambda i: (i, 0)),
            pl.BlockSpec((blk, 16), lambda i: (i, 0)),
        ],
        out_shape=[
            jax.ShapeDtypeStruct((NP, 16), jnp.float32),
            jax.ShapeDtypeStruct((NP, 16), jnp.float32),
            jax.ShapeDtypeStruct((NP, 16), jnp.float32),
        ],
    )(o_a, o_b, s1p, b1r, W2, As2, Ad2)


def _tc3_body(o_ref, s_ref, b2_ref, wo_ref, bo_ref, out_ref):
    o = o_ref[0] + o_ref[1]
    r = 1.0 / (jnp.sum(s_ref[...], axis=0)[:, 0:1] + 1e-16)
    hb = _elu(o * r + b2_ref[...])
    out_ref[...] = jnp.dot(
        hb, wo_ref[...], preferred_element_type=jnp.float32) + bo_ref[...]


def _tc3(o_part, s2p, b2r, Wo, bor):
    blk = 1024
    return pl.pallas_call(
        _tc3_body,
        grid=(NP // blk,),
        in_specs=[
            pl.BlockSpec((2, blk, 16), lambda i: (0, i, 0)),
            pl.BlockSpec((NC * NS, blk, 8), lambda i: (0, i, 0)),
            pl.BlockSpec((1, 16), lambda i: (0, 0)),
            pl.BlockSpec((16, 16), lambda i: (0, 0)),
            pl.BlockSpec((1, 16), lambda i: (0, 0)),
        ],
        out_specs=pl.BlockSpec((blk, 16), lambda i: (i, 0)),
        out_shape=jax.ShapeDtypeStruct((NP, 16), jnp.float32),
    )(o_part, s2p, b2r, Wo, bor)


def kernel(x, edge_index, W1, a_src1, a_dst1, b1, W2, a_src2, a_dst2, b2,
           Wo, bo):
    f32 = jnp.float32
    x_pad = jnp.pad(x, ((0, NP - N), (0, 0)))
    loops = jnp.arange(N, dtype=jnp.int32)
    fill = jnp.full((EP - E - N,), N, jnp.int32)
    srcb = jnp.concatenate([edge_index[0].astype(jnp.int32), loops, fill]
                           ).reshape(NC * NS, NBATCH, 128)
    dstb = jnp.concatenate([edge_index[1].astype(jnp.int32), loops, fill]
                           ).reshape(NC * NS, NBATCH, 128)

    rows = jnp.arange(128)
    As16 = jnp.zeros((128, 16), f32).at[rows, rows // 16].set(
        a_src1.reshape(128))
    Ad16 = jnp.zeros((128, 16), f32).at[rows, rows // 16].set(
        a_dst1.reshape(128))
    As2 = jnp.zeros((16, 16), f32).at[:, 0].set(a_src2[0])
    Ad2 = jnp.zeros((16, 16), f32).at[:, 0].set(a_dst2[0])

    hlo, hhi, ast1, adt1 = _tc1(x_pad, W1, As16, Ad16)
    s1p, w1t = _sc_pass1(srcb, dstb, ast1, adt1)
    o1a = _sc_pass2_l1a(srcb, dstb, w1t, hlo)
    o1b = _sc_pass2_l1b(srcb, dstb, w1t, hhi)
    h2, ast2, adt2 = _tc2(o1a, o1b, s1p.reshape(NC * NS, NP, 8),
                          b1.reshape(1, 128), W2, As2, Ad2)
    s2p, w2t = _sc_pass1(srcb, dstb, ast2, adt2)
    o2p = _sc_pass2_l2(srcb, dstb, w2t, h2)
    out_pad = _tc3(o2p, s2p.reshape(NC * NS, NP, 8),
                   b2.reshape(1, 16), Wo, bo.reshape(1, 16))
    return out_pad[:N]


# speedup vs baseline: 1.1273x; 1.0031x over previous
import functools

import jax
import jax.numpy as jnp
from jax import lax
from jax._src.lax import slicing as _lsl
from jax.experimental import pallas as pl
from jax.experimental.pallas import tpu as pltpu
from jax.experimental.pallas import tpu_sc as plsc

N = 10000
E = 320000
NP = 10240
EP = 331776
NBATCH = 81
NC = 2
NS = 16
LANES = 16

_MESH = plsc.VectorSubcoreMesh(
    core_axis_name="c", subcore_axis_name="s", num_cores=NC, num_subcores=NS)
_SC_PARAMS = pltpu.CompilerParams(
    use_tc_tiling_on_sc=False, needs_layout_passes=False)


def _leaky_exp(u, v):
    e = u + v
    return jnp.exp(jnp.maximum(e, 0.2 * e))


_GDN = _lsl.GatherDimensionNumbers(
    offset_dims=(), collapsed_slice_dims=(0,), start_index_map=(0,))


def _lane_bcast(vec, lane):
    return _lsl.gather(
        vec, jnp.full((LANES, 1), lane, jnp.int32), _GDN, (1,),
        mode=_lsl.GatherScatterMode.PROMISE_IN_BOUNDS)


def _make_pass1():
    @functools.partial(
        pl.kernel,
        mesh=_MESH,
        compiler_params=_SC_PARAMS,
        out_type=[
            jax.ShapeDtypeStruct((NC * NS, NP * 8), jnp.float32),
            jax.ShapeDtypeStruct((EP // 128, 128, 16), jnp.float32),
        ],
        scratch_types=[
            pltpu.VMEM((NBATCH, 128), jnp.int32),
            pltpu.VMEM((NBATCH, 128), jnp.int32),
            pltpu.VMEM((2, 128, 16), jnp.float32),
            pltpu.VMEM((2, 128, 16), jnp.float32),
            pltpu.VMEM((NP * 8,), jnp.float32),
            pltpu.SemaphoreType.DMA,
        ],
    )
    def pass1(srcb, dstb, ast, adt, s_out, w_out, srcv, dstv, gs, gd, s_loc,
              sem):
        cid = lax.axis_index("c")
        sid = lax.axis_index("s")
        wid = cid * NS + sid

        pltpu.sync_copy(srcb.at[wid], srcv)
        pltpu.sync_copy(dstb.at[wid], dstv)

        z16 = jnp.zeros((LANES,), jnp.float32)

        def zrow(i, _):
            s_loc[pl.ds(i * LANES, LANES)] = z16
            return 0
        lax.fori_loop(0, NP * 8 // LANES, zrow, 0)

        lanes = lax.iota(jnp.int32, LANES)
        msk = lanes < 8

        def fire(b, p):
            pltpu.async_copy(ast.at[srcv.at[b]], gs.at[p], sem)
            pltpu.async_copy(adt.at[dstv.at[b]], gd.at[p], sem)

        def drain(p):
            pltpu.make_async_copy(ast.at[srcv.at[0]], gs.at[p], sem).wait()
            pltpu.make_async_copy(adt.at[dstv.at[0]], gd.at[p], sem).wait()

        def work(b, p):
            jv = jnp.full((LANES,), b, jnp.int32)

            def edge(ii):
                w = _leaky_exp(gs[p, ii, :], gd[p, ii, :])
                gs[p, ii, :] = w
                db = plsc.load_gather(
                    dstv, [jv, jnp.full((LANES,), ii, jnp.int32)])
                plsc.addupdate_scatter(
                    s_loc, [db * 8 + lanes], w, mask=msk)
            plsc.parallel_loop(0, 128, unroll=4)(edge)
            pltpu.sync_copy(gs.at[p], w_out.at[wid * NBATCH + b])

        fire(0, 0)

        def it(k, _):
            fire(2 * k + 1, 1)
            drain(0)
            work(2 * k, 0)
            fire(2 * k + 2, 0)
            drain(1)
            work(2 * k + 1, 1)
            return 0

        lax.fori_loop(0, (NBATCH - 1) // 2, it, 0)
        drain(0)
        work(NBATCH - 1, 0)
        pltpu.sync_copy(s_loc, s_out.at[wid])

    return pass1


def _make_pass2_half(hsel):
    HW = 64
    NV = 4
    ZCH = 4

    @functools.partial(
        pl.kernel,
        mesh=_MESH,
        compiler_params=_SC_PARAMS,
        out_type=jax.ShapeDtypeStruct((NC, NP, HW), jnp.float32),
        scratch_types=[
            pltpu.VMEM((NBATCH, 128), jnp.int32),
            pltpu.VMEM((NBATCH, 128), jnp.int32),
            pltpu.VMEM((2, 128, 16), jnp.float32),
            pltpu.VMEM((2, 128, HW), jnp.float32),
            pltpu.VMEM((NP // (NS * ZCH), HW), jnp.float32),
            pltpu.VMEM_SHARED((NP, HW), jnp.float32),
            pltpu.SemaphoreType.DMA,
        ],
    )
    def pass2(srcb, dstb, wt, ht, o_out, srcv, dstv, gw, gh, stage,
              o_acc, sem):
        cid = lax.axis_index("c")
        sid = lax.axis_index("s")
        wid = cid * NS + sid
        rows = NP // (NS * ZCH)

        pltpu.sync_copy(srcb.at[wid], srcv)
        pltpu.sync_copy(dstb.at[wid], dstv)

        def zrow(i, _):
            for h in range(NV):
                stage[i, pl.ds(h * LANES, LANES)] = jnp.zeros(
                    (LANES,), jnp.float32)
            return 0
        lax.fori_loop(0, rows, zrow, 0)
        for z in range(ZCH):
            pltpu.sync_copy(
                stage, o_acc.at[pl.ds((sid * ZCH + z) * rows, rows)])
        plsc.subcore_barrier()

        def fire(b, p):
            pltpu.async_copy(wt.at[wid * NBATCH + b], gw.at[p], sem)
            pltpu.async_copy(ht.at[srcv.at[b]], gh.at[p], sem)

        def drain(p):
            pltpu.make_async_copy(wt.at[0], gw.at[p], sem).wait()
            pltpu.make_async_copy(ht.at[srcv.at[0]], gh.at[p], sem).wait()

        def work(b, p):
            def edge(ii):
                w = gw[p, ii, :]
                for h in range(NV):
                    ab = _lane_bcast(w, hsel * NV + h)
                    gh[p, ii, pl.ds(h * LANES, LANES)] = (
                        gh[p, ii, pl.ds(h * LANES, LANES)] * ab)
            plsc.parallel_loop(0, 128, unroll=4)(edge)
            pltpu.sync_copy(gh.at[p], o_acc.at[dstv.at[b]], add=True)

        fire(0, 0)

        def it(k, _):
            fire(2 * k + 1, 1)
            drain(0)
            work(2 * k, 0)
            fire(2 * k + 2, 0)
            drain(1)
            work(2 * k + 1, 1)
            return 0

        lax.fori_loop(0, (NBATCH - 1) // 2, it, 0)
        drain(0)
        work(NBATCH - 1, 0)

        plsc.subcore_barrier()
        for z in range(ZCH):
            r0 = (sid * ZCH + z) * rows
            pltpu.sync_copy(o_acc.at[pl.ds(r0, rows)], stage)
            pltpu.sync_copy(stage, o_out.at[cid, pl.ds(r0, rows)])

    return pass2


def _make_pass2_l2():
    RW = 16

    @functools.partial(
        pl.kernel,
        mesh=_MESH,
        compiler_params=_SC_PARAMS,
        out_type=jax.ShapeDtypeStruct((NC, NP, RW), jnp.float32),
        scratch_types=[
            pltpu.VMEM((NBATCH, 128), jnp.int32),
            pltpu.VMEM((NBATCH, 128), jnp.int32),
            pltpu.VMEM((2, 128, 16), jnp.float32),
            pltpu.VMEM((2, 128, RW), jnp.float32),
            pltpu.VMEM((NP // NS, RW), jnp.float32),
            pltpu.VMEM_SHARED((NP, RW), jnp.float32),
            pltpu.SemaphoreType.DMA,
        ],
    )
    def pass2(srcb, dstb, wt, ht, o_out, srcv, dstv, gw, gh, stage, o_acc,
              sem):
        cid = lax.axis_index("c")
        sid = lax.axis_index("s")
        wid = cid * NS + sid
        rows = NP // NS

        pltpu.sync_copy(srcb.at[wid], srcv)
        pltpu.sync_copy(dstb.at[wid], dstv)

        def zrow(i, _):
            stage[i, :] = jnp.zeros((LANES,), jnp.float32)
            return 0
        lax.fori_loop(0, rows, zrow, 0)
        pltpu.sync_copy(stage, o_acc.at[pl.ds(sid * rows, rows)])
        plsc.subcore_barrier()

        def fire(b, p):
            pltpu.async_copy(wt.at[wid * NBATCH + b], gw.at[p], sem)
            pltpu.async_copy(ht.at[srcv.at[b]], gh.at[p], sem)

        def drain(p):
            pltpu.make_async_copy(wt.at[0], gw.at[p], sem).wait()
            pltpu.make_async_copy(ht.at[srcv.at[0]], gh.at[p], sem).wait()

        def work(b, p):
            def edge(ii):
                ab = _lane_bcast(gw[p, ii, :], 0)
                gh[p, ii, :] = gh[p, ii, :] * ab
            plsc.parallel_loop(0, 128, unroll=4)(edge)
            pltpu.sync_copy(gh.at[p], o_acc.at[dstv.at[b]], add=True)

        fire(0, 0)

        def it(k, _):
            fire(2 * k + 1, 1)
            drain(0)
            work(2 * k, 0)
            fire(2 * k + 2, 0)
            drain(1)
            work(2 * k + 1, 1)
            return 0

        lax.fori_loop(0, (NBATCH - 1) // 2, it, 0)
        drain(0)
        work(NBATCH - 1, 0)

        plsc.subcore_barrier()
        pltpu.sync_copy(o_acc.at[pl.ds(sid * rows, rows)], stage)
        pltpu.sync_copy(stage, o_out.at[cid, pl.ds(sid * rows, rows)])

    return pass2


_sc_pass1 = _make_pass1()
_sc_pass2_l1a = _make_pass2_half(0)
_sc_pass2_l1b = _make_pass2_half(1)
_sc_pass2_l2 = _make_pass2_l2()


def _tc1_body(x_ref, w1_ref, as_ref, ad_ref, hlo_ref, hhi_ref, ast_ref,
              adt_ref):
    h = jnp.dot(x_ref[...], w1_ref[...], preferred_element_type=jnp.float32)
    hlo_ref[...] = h[:, :64]
    hhi_ref[...] = h[:, 64:]
    ast_ref[...] = jnp.dot(h, as_ref[...], preferred_element_type=jnp.float32)
    adt_ref[...] = jnp.dot(h, ad_ref[...], preferred_element_type=jnp.float32)


def _tc1(x_pad, W1, As16, Ad16):
    blk = 1024
    return pl.pallas_call(
        _tc1_body,
        grid=(NP // blk,),
        in_specs=[
            pl.BlockSpec((blk, 128), lambda i: (i, 0)),
            pl.BlockSpec((128, 128), lambda i: (0, 0)),
            pl.BlockSpec((128, 16), lambda i: (0, 0)),
            pl.BlockSpec((128, 16), lambda i: (0, 0)),
        ],
        out_specs=[
            pl.BlockSpec((blk, 64), lambda i: (i, 0)),
            pl.BlockSpec((blk, 64), lambda i: (i, 0)),
            pl.BlockSpec((blk, 16), lambda i: (i, 0)),
            pl.BlockSpec((blk, 16), lambda i: (i, 0)),
        ],
        out_shape=[
            jax.ShapeDtypeStruct((NP, 64), jnp.float32),
            jax.ShapeDtypeStruct((NP, 64), jnp.float32),
            jax.ShapeDtypeStruct((NP, 16), jnp.float32),
            jax.ShapeDtypeStruct((NP, 16), jnp.float32),
        ],
    )(x_pad, W1, As16, Ad16)


def _elu(x):
    return jnp.where(x > 0, x, jnp.exp(x) - 1.0)


def _tc2_body(oa_ref, ob_ref, s_ref, b1_ref, w2_ref, as_ref, ad_ref, h2_ref,
              ast_ref, adt_ref):
    o = jnp.concatenate([oa_ref[0] + oa_ref[1], ob_ref[0] + ob_ref[1]],
                        axis=-1)
    blk = o.shape[0]
    r = 1.0 / (jnp.sum(s_ref[...], axis=0) + 1e-16)
    rexp = jnp.broadcast_to(r[:, :, None], (blk, 8, 16)).reshape(blk, 128)
    hb = _elu(o * rexp + b1_ref[...])
    h2 = jnp.dot(hb, w2_ref[...], preferred_element_type=jnp.float32)
    h2_ref[...] = h2
    ast_ref[...] = jnp.dot(h2, as_ref[...], preferred_element_type=jnp.float32)
    adt_ref[...] = jnp.dot(h2, ad_ref[...], preferred_element_type=jnp.float32)


def _tc2(o_a, o_b, s1p, b1r, W2, As2, Ad2):
    blk = 1024
    return pl.pallas_call(
        _tc2_body,
        grid=(NP // blk,),
        in_specs=[
            pl.BlockSpec((2, blk, 64), lambda i: (0, i, 0)),
            pl.BlockSpec((2, blk, 64), lambda i: (0, i, 0)),
            pl.BlockSpec((NC * NS, blk, 8), lambda i: (0, i, 0)),
            pl.BlockSpec((1, 128), lambda i: (0, 0)),
            pl.BlockSpec((128, 16), lambda i: (0, 0)),
            pl.BlockSpec((16, 16), lambda i: (0, 0)),
            pl.BlockSpec((16, 16), lambda i: (0, 0)),
        ],
        out_specs=[
            pl.BlockSpec((blk, 16), lambda i: (i, 0)),
            pl.BlockSpec((blk, 16), lambda i: (i, 0)),
            pl.BlockSpec((blk, 16), lambda i: (i, 0)),
        ],
        out_shape=[
            jax.ShapeDtypeStruct((NP, 16), jnp.float32),
            jax.ShapeDtypeStruct((NP, 16), jnp.float32),
            jax.ShapeDtypeStruct((NP, 16), jnp.float32),
        ],
    )(o_a, o_b, s1p, b1r, W2, As2, Ad2)


def _tc3_body(o_ref, s_ref, b2_ref, wo_ref, bo_ref, out_ref):
    o = o_ref[0] + o_ref[1]
    r = 1.0 / (jnp.sum(s_ref[...], axis=0)[:, 0:1] + 1e-16)
    hb = _elu(o * r + b2_ref[...])
    out_ref[...] = jnp.dot(
        hb, wo_ref[...], preferred_element_type=jnp.float32) + bo_ref[...]


def _tc3(o_part, s2p, b2r, Wo, bor):
    blk = 1024
    return pl.pallas_call(
        _tc3_body,
        grid=(NP // blk,),
        in_specs=[
            pl.BlockSpec((2, blk, 16), lambda i: (0, i, 0)),
            pl.BlockSpec((NC * NS, blk, 8), lambda i: (0, i, 0)),
            pl.BlockSpec((1, 16), lambda i: (0, 0)),
            pl.BlockSpec((16, 16), lambda i: (0, 0)),
            pl.BlockSpec((1, 16), lambda i: (0, 0)),
        ],
        out_specs=pl.BlockSpec((blk, 16), lambda i: (i, 0)),
        out_shape=jax.ShapeDtypeStruct((NP, 16), jnp.float32),
    )(o_part, s2p, b2r, Wo, bor)


def kernel(x, edge_index, W1, a_src1, a_dst1, b1, W2, a_src2, a_dst2, b2,
           Wo, bo):
    f32 = jnp.float32
    x_pad = jnp.pad(x, ((0, NP - N), (0, 0)))
    loops = jnp.arange(N, dtype=jnp.int32)
    fill = jnp.full((EP - E - N,), N, jnp.int32)
    srcb = jnp.concatenate([edge_index[0].astype(jnp.int32), loops, fill]
                           ).reshape(NC * NS, NBATCH, 128)
    dstb = jnp.concatenate([edge_index[1].astype(jnp.int32), loops, fill]
                           ).reshape(NC * NS, NBATCH, 128)

    rows = jnp.arange(128)
    As16 = jnp.zeros((128, 16), f32).at[rows, rows // 16].set(
        a_src1.reshape(128))
    Ad16 = jnp.zeros((128, 16), f32).at[rows, rows // 16].set(
        a_dst1.reshape(128))
    As2 = jnp.zeros((16, 16), f32).at[:, 0].set(a_src2[0])
    Ad2 = jnp.zeros((16, 16), f32).at[:, 0].set(a_dst2[0])

    hlo, hhi, ast1, adt1 = _tc1(x_pad, W1, As16, Ad16)
    s1p, w1t = _sc_pass1(srcb, dstb, ast1, adt1)
    o1a = _sc_pass2_l1a(srcb, dstb, w1t, hlo)
    o1b = _sc_pass2_l1b(srcb, dstb, w1t, hhi)
    h2, ast2, adt2 = _tc2(o1a, o1b, s1p.reshape(NC * NS, NP, 8),
                          b1.reshape(1, 128), W2, As2, Ad2)
    s2p, w2t = _sc_pass1(srcb, dstb, ast2, adt2)
    o2p = _sc_pass2_l2(srcb, dstb, w2t, h2)
    out_pad = _tc3(o2p, s2p.reshape(NC * NS, NP, 8),
                   b2.reshape(1, 16), Wo, bo.reshape(1, 16))
    return out_pad[:N]
